# Initial kernel scaffold; baseline (speedup 1.0000x reference)
import jax
import jax.numpy as jnp
from jax.experimental import pallas as pl


def kernel(x, edge_index, W1, b1, g1, bt1, W2, b2, g2, bt2, W3, b3, g3, bt3):
    raise NotImplementedError("write your pallas kernel here")


# speedup vs baseline: 17.3774x; 17.3774x over previous
import functools

import jax
import jax.numpy as jnp
from jax import lax
from jax.experimental import pallas as pl
from jax.experimental.pallas import tpu as pltpu
from jax.experimental.pallas import tpu_sc as plsc

N = 10000
N_PAD = 10240
D = 128
E = 320000

NC = 2
NS = 16
NW = NC * NS
K = 125
CHUNKS = E // K
CHUNKS_PER_W = CHUNKS // NW
EDGES_PER_W = E // NW
ROWS_PER_TILE = N_PAD // NS
ROW_BLK = 1024

_mesh = plsc.VectorSubcoreMesh(core_axis_name="c", subcore_axis_name="s")
_sc_params = pltpu.CompilerParams(needs_layout_passes=False)


@functools.partial(
    pl.kernel,
    out_type=jax.ShapeDtypeStruct((NW * N_PAD,), jnp.float32),
    mesh=_mesh,
    scratch_types=[
        pltpu.VMEM((EDGES_PER_W,), jnp.int32),
        pltpu.VMEM((N_PAD,), jnp.float32),
    ],
    compiler_params=_sc_params,
)
def _sc_hist(dst_hbm, out_hbm, dst_v, hist_v):
    c = lax.axis_index("c")
    s = lax.axis_index("s")
    wid = s * NC + c
    pltpu.sync_copy(dst_hbm.at[pl.ds(wid * EDGES_PER_W, EDGES_PER_W)], dst_v)

    zero16 = jnp.zeros((16,), jnp.float32)

    def zbody(i, carry):
        hist_v[pl.ds(i * 16, 16)] = zero16
        return carry

    lax.fori_loop(0, N_PAD // 16, zbody, 0, unroll=4)

    ones16 = jnp.ones((16,), jnp.float32)

    def body(i, carry):
        idx = dst_v[pl.ds(i * 16, 16)]
        plsc.addupdate_scatter(hist_v, [idx], ones16)
        return carry

    lax.fori_loop(0, EDGES_PER_W // 16, body, 0, unroll=4)
    pltpu.sync_copy(hist_v, out_hbm.at[pl.ds(wid * N_PAD, N_PAD)])


@functools.partial(
    pl.kernel,
    out_type=jax.ShapeDtypeStruct((NC, N_PAD, D), jnp.float32),
    mesh=_mesh,
    scratch_types=[
        pltpu.VMEM((CHUNKS_PER_W, K), jnp.int32),
        pltpu.VMEM((CHUNKS_PER_W, K), jnp.int32),
        pltpu.VMEM((K, D), jnp.float32),
        pltpu.VMEM_SHARED((N_PAD, D), jnp.float32),
        pltpu.SemaphoreType.DMA,
    ],
    compiler_params=_sc_params,
)
def _sc_scatter(h_hbm, src_hbm, dst_hbm, zeros_hbm, part_hbm,
                src_v, dst_v, rows_v, acc_sh, sem):
    c = lax.axis_index("c")
    s = lax.axis_index("s")
    wid = s * NC + c

    pltpu.sync_copy(src_hbm.at[pl.ds(wid * CHUNKS_PER_W, CHUNKS_PER_W)], src_v)
    pltpu.sync_copy(dst_hbm.at[pl.ds(wid * CHUNKS_PER_W, CHUNKS_PER_W)], dst_v)
    pltpu.sync_copy(zeros_hbm, acc_sh.at[pl.ds(s * ROWS_PER_TILE, ROWS_PER_TILE)])
    plsc.subcore_barrier()

    def chunk(j, carry):
        pltpu.async_copy(h_hbm.at[src_v.at[j]], rows_v, sem).wait()
        pltpu.sync_copy(rows_v, acc_sh.at[dst_v.at[j]], add=True)
        return carry

    lax.fori_loop(0, CHUNKS_PER_W, chunk, 0)
    plsc.subcore_barrier()
    pltpu.sync_copy(
        acc_sh.at[pl.ds(s * ROWS_PER_TILE, ROWS_PER_TILE)],
        part_hbm.at[c, pl.ds(s * ROWS_PER_TILE, ROWS_PER_TILE)],
    )


def _prep_body(hist_ref, x_ref, w_ref, dinv_ref, h_ref):
    deg = jnp.sum(hist_ref[...], axis=0) + 1.0
    dinv = lax.rsqrt(deg)
    h = lax.dot_general(x_ref[...], w_ref[...], (((1,), (1,)), ((), ())),
                        preferred_element_type=jnp.float32)
    dinv_ref[...] = dinv[:, None]
    h_ref[...] = h * dinv[:, None]


def _tc_prep(hist, x, w):
    grid = (N_PAD // ROW_BLK,)
    return pl.pallas_call(
        _prep_body,
        grid=grid,
        in_specs=[
            pl.BlockSpec((NW, ROW_BLK), lambda i: (0, i)),
            pl.BlockSpec((ROW_BLK, D), lambda i: (i, 0)),
            pl.BlockSpec((D, D), lambda i: (0, 0)),
        ],
        out_specs=[
            pl.BlockSpec((ROW_BLK, 1), lambda i: (i, 0)),
            pl.BlockSpec((ROW_BLK, D), lambda i: (i, 0)),
        ],
        out_shape=[
            jax.ShapeDtypeStruct((N_PAD, 1), jnp.float32),
            jax.ShapeDtypeStruct((N_PAD, D), jnp.float32),
        ],
    )(hist, x, w)


def _layer_norm(h, g, b):
    mu = jnp.mean(h, axis=-1, keepdims=True)
    var = jnp.mean((h - mu) ** 2, axis=-1, keepdims=True)
    return (h - mu) * lax.rsqrt(var + 1e-5) * g + b


def _mid_body(p0_ref, p1_ref, hs_ref, dinv_ref, res_ref, b_ref, g_ref,
              bt_ref, w_ref, out_ref, hn_ref):
    dinv = dinv_ref[...]
    agg = (p0_ref[...] + p1_ref[...] + hs_ref[...]) * dinv + b_ref[...]
    ln = _layer_norm(agg, g_ref[...], bt_ref[...])
    res = jnp.maximum(ln, 0.0) + res_ref[...]
    out_ref[...] = res
    hn = lax.dot_general(res, w_ref[...], (((1,), (1,)), ((), ())),
                         preferred_element_type=jnp.float32)
    hn_ref[...] = hn * dinv


def _tc_mid(p0, p1, hs, dinv, res, b, g, bt, w):
    grid = (N_PAD // ROW_BLK,)
    row = pl.BlockSpec((ROW_BLK, D), lambda i: (i, 0))
    vec = pl.BlockSpec((1, D), lambda i: (0, 0))
    return pl.pallas_call(
        _mid_body,
        grid=grid,
        in_specs=[row, row, row,
                  pl.BlockSpec((ROW_BLK, 1), lambda i: (i, 0)),
                  row, vec, vec, vec,
                  pl.BlockSpec((D, D), lambda i: (0, 0))],
        out_specs=[row, row],
        out_shape=[
            jax.ShapeDtypeStruct((N_PAD, D), jnp.float32),
            jax.ShapeDtypeStruct((N_PAD, D), jnp.float32),
        ],
    )(p0, p1, hs, dinv, res, b[None, :], g[None, :], bt[None, :], w)


def _final_body(p0_ref, p1_ref, hs_ref, dinv_ref, b_ref, g_ref, bt_ref,
                out_ref):
    agg = (p0_ref[...] + p1_ref[...] + hs_ref[...]) * dinv_ref[...] + b_ref[...]
    out_ref[...] = _layer_norm(agg, g_ref[...], bt_ref[...])


def _tc_final(p0, p1, hs, dinv, b, g, bt):
    grid = (N_PAD // ROW_BLK,)
    row = pl.BlockSpec((ROW_BLK, D), lambda i: (i, 0))
    vec = pl.BlockSpec((1, D), lambda i: (0, 0))
    return pl.pallas_call(
        _final_body,
        grid=grid,
        in_specs=[row, row, row,
                  pl.BlockSpec((ROW_BLK, 1), lambda i: (i, 0)),
                  vec, vec, vec],
        out_specs=row,
        out_shape=jax.ShapeDtypeStruct((N_PAD, D), jnp.float32),
    )(p0, p1, hs, dinv, b[None, :], g[None, :], bt[None, :])


def kernel(x, edge_index, W1, b1, g1, bt1, W2, b2, g2, bt2, W3, b3, g3, bt3):
    src2 = edge_index[0].reshape(CHUNKS, K)
    dst = edge_index[1]
    dst2 = dst.reshape(CHUNKS, K)
    zeros = jnp.zeros((ROWS_PER_TILE, D), jnp.float32)
    x_pad = jnp.pad(x, ((0, N_PAD - N), (0, 0)))

    hist = _sc_hist(dst).reshape(NW, N_PAD)
    dinv, h1 = _tc_prep(hist, x_pad, W1)

    parts = _sc_scatter(h1, src2, dst2, zeros)
    res1, h2 = _tc_mid(parts[0], parts[1], h1, dinv, x_pad, b1, g1, bt1, W2)

    parts = _sc_scatter(h2, src2, dst2, zeros)
    res2, h3 = _tc_mid(parts[0], parts[1], h2, dinv, res1, b2, g2, bt2, W3)

    parts = _sc_scatter(h3, src2, dst2, zeros)
    out = _tc_final(parts[0], parts[1], h3, dinv, b3, g3, bt3)
    return out[:N]


# speedup vs baseline: 21.7445x; 1.2513x over previous
import functools

import jax
import jax.numpy as jnp
from jax import lax
from jax.experimental import pallas as pl
from jax.experimental.pallas import tpu as pltpu
from jax.experimental.pallas import tpu_sc as plsc

N = 10000
N_PAD = 10240
D = 128
E = 320000

NC = 2
NS = 16
NW = NC * NS
K = 125
CHUNKS = E // K
CHUNKS_PER_W = CHUNKS // NW
IBLK = 40
EDGES_PER_W = E // NW
ROWS_PER_TILE = N_PAD // NS
ROW_BLK = 1024

_mesh = plsc.VectorSubcoreMesh(core_axis_name="c", subcore_axis_name="s")
_sc_params = pltpu.CompilerParams(needs_layout_passes=False)


@functools.partial(
    pl.kernel,
    out_type=jax.ShapeDtypeStruct((NW * N_PAD,), jnp.float32),
    mesh=_mesh,
    scratch_types=[
        pltpu.VMEM((EDGES_PER_W,), jnp.int32),
        pltpu.VMEM((N_PAD,), jnp.float32),
    ],
    compiler_params=_sc_params,
)
def _sc_hist(dst_hbm, out_hbm, dst_v, hist_v):
    c = lax.axis_index("c")
    s = lax.axis_index("s")
    wid = s * NC + c
    pltpu.sync_copy(dst_hbm.at[pl.ds(wid * EDGES_PER_W, EDGES_PER_W)], dst_v)

    zero16 = jnp.zeros((16,), jnp.float32)

    def zbody(i, carry):
        hist_v[pl.ds(i * 16, 16)] = zero16
        return carry

    lax.fori_loop(0, N_PAD // 16, zbody, 0, unroll=4)

    ones16 = jnp.ones((16,), jnp.float32)

    def body(i, carry):
        idx = dst_v[pl.ds(i * 16, 16)]
        plsc.addupdate_scatter(hist_v, [idx], ones16)
        return carry

    lax.fori_loop(0, EDGES_PER_W // 16, body, 0, unroll=4)
    pltpu.sync_copy(hist_v, out_hbm.at[pl.ds(wid * N_PAD, N_PAD)])


@functools.partial(
    pl.kernel,
    out_type=jax.ShapeDtypeStruct((NC, N_PAD, D), jnp.float32),
    mesh=_mesh,
    scratch_types=[
        pltpu.VMEM((IBLK, K), jnp.int32),
        pltpu.VMEM((IBLK, K), jnp.int32),
        pltpu.VMEM((K, D), jnp.float32),
        pltpu.VMEM((K, D), jnp.float32),
        pltpu.VMEM_SHARED((N_PAD, D), jnp.float32),
        pltpu.SemaphoreType.DMA,
        pltpu.SemaphoreType.DMA,
        pltpu.SemaphoreType.DMA,
        pltpu.SemaphoreType.DMA,
    ],
    compiler_params=_sc_params,
)
def _sc_scatter(h_hbm, src_hbm, dst_hbm, zeros_hbm, part_hbm,
                src_v, dst_v, rows_a, rows_b, acc_sh, sga, sgb, ssa, ssb):
    c = lax.axis_index("c")
    s = lax.axis_index("s")
    wid = s * NC + c

    pltpu.sync_copy(zeros_hbm, acc_sh.at[pl.ds(s * ROWS_PER_TILE, ROWS_PER_TILE)])
    plsc.subcore_barrier()

    npair = IBLK // 2

    def blk(b, carry):
        base = wid * CHUNKS_PER_W + b * IBLK
        pltpu.sync_copy(src_hbm.at[pl.ds(base, IBLK)], src_v)
        pltpu.sync_copy(dst_hbm.at[pl.ds(base, IBLK)], dst_v)
        pltpu.async_copy(h_hbm.at[src_v.at[0]], rows_a, sga)

        def pair(i, carry2):
            j = 2 * i
            pltpu.make_async_copy(h_hbm.at[src_v.at[j]], rows_a, sga).wait()
            pltpu.async_copy(h_hbm.at[src_v.at[j + 1]], rows_b, sgb)
            scat_a = pltpu.async_copy(rows_a, acc_sh.at[dst_v.at[j]], ssa,
                                      add=True)
            pltpu.make_async_copy(h_hbm.at[src_v.at[j + 1]], rows_b, sgb).wait()
            scat_a.wait()

            @pl.when(i < npair - 1)
            def _():
                pltpu.async_copy(h_hbm.at[src_v.at[j + 2]], rows_a, sga)

            pltpu.async_copy(rows_b, acc_sh.at[dst_v.at[j + 1]], ssb,
                             add=True).wait()
            return carry2

        lax.fori_loop(0, npair, pair, 0)
        return carry

    lax.fori_loop(0, CHUNKS_PER_W // IBLK, blk, 0)
    plsc.subcore_barrier()
    pltpu.sync_copy(
        acc_sh.at[pl.ds(s * ROWS_PER_TILE, ROWS_PER_TILE)],
        part_hbm.at[c, pl.ds(s * ROWS_PER_TILE, ROWS_PER_TILE)],
    )


def _prep_body(hist_ref, x_ref, w_ref, dinv_ref, h_ref):
    deg = jnp.sum(hist_ref[...], axis=0) + 1.0
    dinv = lax.rsqrt(deg)
    h = lax.dot_general(x_ref[...], w_ref[...], (((1,), (1,)), ((), ())),
                        preferred_element_type=jnp.float32)
    dinv_ref[...] = dinv[:, None]
    h_ref[...] = h * dinv[:, None]


def _tc_prep(hist, x, w):
    grid = (N_PAD // ROW_BLK,)
    return pl.pallas_call(
        _prep_body,
        grid=grid,
        in_specs=[
            pl.BlockSpec((NW, ROW_BLK), lambda i: (0, i)),
            pl.BlockSpec((ROW_BLK, D), lambda i: (i, 0)),
            pl.BlockSpec((D, D), lambda i: (0, 0)),
        ],
        out_specs=[
            pl.BlockSpec((ROW_BLK, 1), lambda i: (i, 0)),
            pl.BlockSpec((ROW_BLK, D), lambda i: (i, 0)),
        ],
        out_shape=[
            jax.ShapeDtypeStruct((N_PAD, 1), jnp.float32),
            jax.ShapeDtypeStruct((N_PAD, D), jnp.float32),
        ],
    )(hist, x, w)


def _layer_norm(h, g, b):
    mu = jnp.mean(h, axis=-1, keepdims=True)
    var = jnp.mean((h - mu) ** 2, axis=-1, keepdims=True)
    return (h - mu) * lax.rsqrt(var + 1e-5) * g + b


def _mid_body(p0_ref, p1_ref, hs_ref, dinv_ref, res_ref, b_ref, g_ref,
              bt_ref, w_ref, out_ref, hn_ref):
    dinv = dinv_ref[...]
    agg = (p0_ref[...] + p1_ref[...] + hs_ref[...]) * dinv + b_ref[...]
    ln = _layer_norm(agg, g_ref[...], bt_ref[...])
    res = jnp.maximum(ln, 0.0) + res_ref[...]
    out_ref[...] = res
    hn = lax.dot_general(res, w_ref[...], (((1,), (1,)), ((), ())),
                         preferred_element_type=jnp.float32)
    hn_ref[...] = hn * dinv


def _tc_mid(p0, p1, hs, dinv, res, b, g, bt, w):
    grid = (N_PAD // ROW_BLK,)
    row = pl.BlockSpec((ROW_BLK, D), lambda i: (i, 0))
    vec = pl.BlockSpec((1, D), lambda i: (0, 0))
    return pl.pallas_call(
        _mid_body,
        grid=grid,
        in_specs=[row, row, row,
                  pl.BlockSpec((ROW_BLK, 1), lambda i: (i, 0)),
                  row, vec, vec, vec,
                  pl.BlockSpec((D, D), lambda i: (0, 0))],
        out_specs=[row, row],
        out_shape=[
            jax.ShapeDtypeStruct((N_PAD, D), jnp.float32),
            jax.ShapeDtypeStruct((N_PAD, D), jnp.float32),
        ],
    )(p0, p1, hs, dinv, res, b[None, :], g[None, :], bt[None, :], w)


def _final_body(p0_ref, p1_ref, hs_ref, dinv_ref, b_ref, g_ref, bt_ref,
                out_ref):
    agg = (p0_ref[...] + p1_ref[...] + hs_ref[...]) * dinv_ref[...] + b_ref[...]
    out_ref[...] = _layer_norm(agg, g_ref[...], bt_ref[...])


def _tc_final(p0, p1, hs, dinv, b, g, bt):
    grid = (N_PAD // ROW_BLK,)
    row = pl.BlockSpec((ROW_BLK, D), lambda i: (i, 0))
    vec = pl.BlockSpec((1, D), lambda i: (0, 0))
    return pl.pallas_call(
        _final_body,
        grid=grid,
        in_specs=[row, row, row,
                  pl.BlockSpec((ROW_BLK, 1), lambda i: (i, 0)),
                  vec, vec, vec],
        out_specs=row,
        out_shape=jax.ShapeDtypeStruct((N_PAD, D), jnp.float32),
    )(p0, p1, hs, dinv, b[None, :], g[None, :], bt[None, :])


def kernel(x, edge_index, W1, b1, g1, bt1, W2, b2, g2, bt2, W3, b3, g3, bt3):
    src2 = edge_index[0].reshape(CHUNKS, K)
    dst = edge_index[1]
    dst2 = dst.reshape(CHUNKS, K)
    zeros = jnp.zeros((ROWS_PER_TILE, D), jnp.float32)
    x_pad = jnp.pad(x, ((0, N_PAD - N), (0, 0)))

    hist = _sc_hist(dst).reshape(NW, N_PAD)
    dinv, h1 = _tc_prep(hist, x_pad, W1)

    parts = _sc_scatter(h1, src2, dst2, zeros)
    res1, h2 = _tc_mid(parts[0], parts[1], h1, dinv, x_pad, b1, g1, bt1, W2)

    parts = _sc_scatter(h2, src2, dst2, zeros)
    res2, h3 = _tc_mid(parts[0], parts[1], h2, dinv, res1, b2, g2, bt2, W3)

    parts = _sc_scatter(h3, src2, dst2, zeros)
    out = _tc_final(parts[0], parts[1], h3, dinv, b3, g3, bt3)
    return out[:N]


# speedup vs baseline: 24.9871x; 1.1491x over previous
import functools

import jax
import jax.numpy as jnp
from jax import lax
from jax.experimental import pallas as pl
from jax.experimental.pallas import tpu as pltpu
from jax.experimental.pallas import tpu_sc as plsc

N = 10000
N_PAD = 10240
D = 128
E = 320000

NC = 2
NS = 16
NW = NC * NS
K = 50
CHUNKS = E // K
CHUNKS_PER_W = CHUNKS // NW
IBLK = 40
NBUF = 5
EDGES_PER_W = E // NW
ROWS_PER_TILE = N_PAD // NS
ROW_BLK = 1024

_mesh = plsc.VectorSubcoreMesh(core_axis_name="c", subcore_axis_name="s")
_sc_params = pltpu.CompilerParams(needs_layout_passes=False)


@functools.partial(
    pl.kernel,
    out_type=jax.ShapeDtypeStruct((NW * N_PAD,), jnp.float32),
    mesh=_mesh,
    scratch_types=[
        pltpu.VMEM((EDGES_PER_W,), jnp.int32),
        pltpu.VMEM((N_PAD,), jnp.float32),
    ],
    compiler_params=_sc_params,
)
def _sc_hist(dst_hbm, out_hbm, dst_v, hist_v):
    c = lax.axis_index("c")
    s = lax.axis_index("s")
    wid = s * NC + c
    pltpu.sync_copy(dst_hbm.at[pl.ds(wid * EDGES_PER_W, EDGES_PER_W)], dst_v)

    zero16 = jnp.zeros((16,), jnp.float32)

    def zbody(i, carry):
        hist_v[pl.ds(i * 16, 16)] = zero16
        return carry

    lax.fori_loop(0, N_PAD // 16, zbody, 0, unroll=4)

    ones16 = jnp.ones((16,), jnp.float32)

    def body(i, carry):
        idx = dst_v[pl.ds(i * 16, 16)]
        plsc.addupdate_scatter(hist_v, [idx], ones16)
        return carry

    lax.fori_loop(0, EDGES_PER_W // 16, body, 0, unroll=4)
    pltpu.sync_copy(hist_v, out_hbm.at[pl.ds(wid * N_PAD, N_PAD)])


@functools.partial(
    pl.kernel,
    out_type=jax.ShapeDtypeStruct((NC, N_PAD, D), jnp.float32),
    mesh=_mesh,
    scratch_types=[
        pltpu.VMEM((IBLK, K), jnp.int32),
        pltpu.VMEM((IBLK, K), jnp.int32),
        [pltpu.VMEM((K, D), jnp.float32)] * NBUF,
        pltpu.VMEM_SHARED((N_PAD, D), jnp.float32),
        [pltpu.SemaphoreType.DMA] * NBUF,
        [pltpu.SemaphoreType.DMA] * NBUF,
    ],
    compiler_params=_sc_params,
)
def _sc_scatter(h_hbm, src_hbm, dst_hbm, zeros_hbm, part_hbm,
                src_v, dst_v, rows, acc_sh, sg, ss):
    c = lax.axis_index("c")
    s = lax.axis_index("s")
    wid = s * NC + c

    pltpu.sync_copy(zeros_hbm, acc_sh.at[pl.ds(s * ROWS_PER_TILE, ROWS_PER_TILE)])
    plsc.subcore_barrier()

    def blk(b, carry):
        base = wid * CHUNKS_PER_W + b * IBLK
        pltpu.sync_copy(src_hbm.at[pl.ds(base, IBLK)], src_v)
        pltpu.sync_copy(dst_hbm.at[pl.ds(base, IBLK)], dst_v)
        for p in range(NBUF):
            pltpu.async_copy(h_hbm.at[src_v.at[p]], rows[p], sg[p])

        def ring(i, carry2):
            for p in range(NBUF):
                j = i * NBUF + p
                pltpu.make_async_copy(h_hbm.at[src_v.at[j]], rows[p],
                                      sg[p]).wait()
                pltpu.async_copy(rows[p], acc_sh.at[dst_v.at[j]], ss[p],
                                 add=True).wait()

                @pl.when(j + NBUF < IBLK)
                def _():
                    pltpu.async_copy(h_hbm.at[src_v.at[j + NBUF]], rows[p],
                                     sg[p])
            return carry2

        lax.fori_loop(0, IBLK // NBUF, ring, 0)
        return carry

    lax.fori_loop(0, CHUNKS_PER_W // IBLK, blk, 0)
    plsc.subcore_barrier()
    pltpu.sync_copy(
        acc_sh.at[pl.ds(s * ROWS_PER_TILE, ROWS_PER_TILE)],
        part_hbm.at[c, pl.ds(s * ROWS_PER_TILE, ROWS_PER_TILE)],
    )


def _prep_body(hist_ref, x_ref, w_ref, dinv_ref, h_ref):
    deg = jnp.sum(hist_ref[...], axis=0) + 1.0
    dinv = lax.rsqrt(deg)
    h = lax.dot_general(x_ref[...], w_ref[...], (((1,), (1,)), ((), ())),
                        preferred_element_type=jnp.float32)
    dinv_ref[...] = dinv[:, None]
    h_ref[...] = h * dinv[:, None]


def _tc_prep(hist, x, w):
    grid = (N_PAD // ROW_BLK,)
    return pl.pallas_call(
        _prep_body,
        grid=grid,
        in_specs=[
            pl.BlockSpec((NW, ROW_BLK), lambda i: (0, i)),
            pl.BlockSpec((ROW_BLK, D), lambda i: (i, 0)),
            pl.BlockSpec((D, D), lambda i: (0, 0)),
        ],
        out_specs=[
            pl.BlockSpec((ROW_BLK, 1), lambda i: (i, 0)),
            pl.BlockSpec((ROW_BLK, D), lambda i: (i, 0)),
        ],
        out_shape=[
            jax.ShapeDtypeStruct((N_PAD, 1), jnp.float32),
            jax.ShapeDtypeStruct((N_PAD, D), jnp.float32),
        ],
    )(hist, x, w)


def _layer_norm(h, g, b):
    mu = jnp.mean(h, axis=-1, keepdims=True)
    var = jnp.mean((h - mu) ** 2, axis=-1, keepdims=True)
    return (h - mu) * lax.rsqrt(var + 1e-5) * g + b


def _mid_body(p0_ref, p1_ref, hs_ref, dinv_ref, res_ref, b_ref, g_ref,
              bt_ref, w_ref, out_ref, hn_ref):
    dinv = dinv_ref[...]
    agg = (p0_ref[...] + p1_ref[...] + hs_ref[...]) * dinv + b_ref[...]
    ln = _layer_norm(agg, g_ref[...], bt_ref[...])
    res = jnp.maximum(ln, 0.0) + res_ref[...]
    out_ref[...] = res
    hn = lax.dot_general(res, w_ref[...], (((1,), (1,)), ((), ())),
                         preferred_element_type=jnp.float32)
    hn_ref[...] = hn * dinv


def _tc_mid(p0, p1, hs, dinv, res, b, g, bt, w):
    grid = (N_PAD // ROW_BLK,)
    row = pl.BlockSpec((ROW_BLK, D), lambda i: (i, 0))
    vec = pl.BlockSpec((1, D), lambda i: (0, 0))
    return pl.pallas_call(
        _mid_body,
        grid=grid,
        in_specs=[row, row, row,
                  pl.BlockSpec((ROW_BLK, 1), lambda i: (i, 0)),
                  row, vec, vec, vec,
                  pl.BlockSpec((D, D), lambda i: (0, 0))],
        out_specs=[row, row],
        out_shape=[
            jax.ShapeDtypeStruct((N_PAD, D), jnp.float32),
            jax.ShapeDtypeStruct((N_PAD, D), jnp.float32),
        ],
    )(p0, p1, hs, dinv, res, b[None, :], g[None, :], bt[None, :], w)


def _final_body(p0_ref, p1_ref, hs_ref, dinv_ref, b_ref, g_ref, bt_ref,
                out_ref):
    agg = (p0_ref[...] + p1_ref[...] + hs_ref[...]) * dinv_ref[...] + b_ref[...]
    out_ref[...] = _layer_norm(agg, g_ref[...], bt_ref[...])


def _tc_final(p0, p1, hs, dinv, b, g, bt):
    grid = (N_PAD // ROW_BLK,)
    row = pl.BlockSpec((ROW_BLK, D), lambda i: (i, 0))
    vec = pl.BlockSpec((1, D), lambda i: (0, 0))
    return pl.pallas_call(
        _final_body,
        grid=grid,
        in_specs=[row, row, row,
                  pl.BlockSpec((ROW_BLK, 1), lambda i: (i, 0)),
                  vec, vec, vec],
        out_specs=row,
        out_shape=jax.ShapeDtypeStruct((N_PAD, D), jnp.float32),
    )(p0, p1, hs, dinv, b[None, :], g[None, :], bt[None, :])


def kernel(x, edge_index, W1, b1, g1, bt1, W2, b2, g2, bt2, W3, b3, g3, bt3):
    src2 = edge_index[0].reshape(CHUNKS, K)
    dst = edge_index[1]
    dst2 = dst.reshape(CHUNKS, K)
    zeros = jnp.zeros((ROWS_PER_TILE, D), jnp.float32)
    x_pad = jnp.pad(x, ((0, N_PAD - N), (0, 0)))

    hist = _sc_hist(dst).reshape(NW, N_PAD)
    dinv, h1 = _tc_prep(hist, x_pad, W1)

    parts = _sc_scatter(h1, src2, dst2, zeros)
    res1, h2 = _tc_mid(parts[0], parts[1], h1, dinv, x_pad, b1, g1, bt1, W2)

    parts = _sc_scatter(h2, src2, dst2, zeros)
    res2, h3 = _tc_mid(parts[0], parts[1], h2, dinv, res1, b2, g2, bt2, W3)

    parts = _sc_scatter(h3, src2, dst2, zeros)
    out = _tc_final(parts[0], parts[1], h3, dinv, b3, g3, bt3)
    return out[:N]
